# Initial kernel scaffold; baseline (speedup 1.0000x reference)
import jax
import jax.numpy as jnp
from jax.experimental import pallas as pl


def kernel(x, edge_index, W1, b1, g1, be1, W2, b2, g2, be2, W3, b3, g3, be3, W4, b4, g4, be4):
    raise NotImplementedError("write your pallas kernel here")


# speedup vs baseline: 4.6609x; 4.6609x over previous
import functools

import jax
import jax.numpy as jnp
from jax import lax
from jax.experimental import pallas as pl
from jax.experimental.pallas import tpu as pltpu, tpu_sc as plsc

N = 10000
E = 320000
D_IN = 128
H = 512
D_OUT = 256

N_PAD = 10240
EROWS = 2560
E_PAD = EROWS * 128
PAD_IDX = N_PAD - 1

NS = 16
ROWS_T = N_PAD // NS
ER_T = EROWS // NS
KB = 40

BN = 512


def _make_deg():
    mesh = plsc.VectorSubcoreMesh(core_axis_name="c", subcore_axis_name="s")
    groups = E_PAD // 16 // 32

    @functools.partial(
        pl.kernel,
        out_type=jax.ShapeDtypeStruct((2, N_PAD), jnp.float32),
        mesh=mesh,
        compiler_params=pltpu.CompilerParams(needs_layout_passes=False),
        scratch_types=[
            pltpu.VMEM((E_PAD // 32,), jnp.int32),
            pltpu.VMEM((N_PAD,), jnp.float32),
            pltpu.VMEM((ROWS_T,), jnp.float32),
            pltpu.VMEM((ROWS_T,), jnp.float32),
            pltpu.VMEM_SHARED((NS, N_PAD), jnp.float32),
        ],
    )
    def deg_kernel(dst_hbm, parts_hbm, idx_v, hist_v, acc_v, tmp_v, sh):
        cid = lax.axis_index("c")
        sid = lax.axis_index("s")
        e0 = (cid * NS + sid) * (E_PAD // 32)

        zero = jnp.zeros((16,), jnp.float32)

        def zbody(i, c):
            hist_v[pl.ds(i * 16, 16)] = zero
            return c

        lax.fori_loop(0, N_PAD // 16, zbody, 0)

        pltpu.sync_copy(dst_hbm.at[pl.ds(e0, E_PAD // 32)], idx_v)
        ones = jnp.ones((16,), jnp.float32)

        def hbody(i, c):
            idx = idx_v[pl.ds(i * 16, 16)]
            plsc.addupdate_scatter(hist_v, [idx], ones)
            return c

        lax.fori_loop(0, groups, hbody, 0)

        pltpu.sync_copy(hist_v, sh.at[sid])
        plsc.subcore_barrier()

        n0 = sid * ROWS_T
        pltpu.sync_copy(sh.at[0, pl.ds(n0, ROWS_T)], acc_v)

        def rbody(k, c):
            pltpu.sync_copy(sh.at[k, pl.ds(n0, ROWS_T)], tmp_v)

            def abody(j, c2):
                acc_v[pl.ds(j * 16, 16)] = (
                    acc_v[pl.ds(j * 16, 16)] + tmp_v[pl.ds(j * 16, 16)]
                )
                return c2

            lax.fori_loop(0, ROWS_T // 16, abody, 0)
            return c

        lax.fori_loop(1, NS, rbody, 0)
        pltpu.sync_copy(acc_v, parts_hbm.at[cid, pl.ds(n0, ROWS_T)])

    return deg_kernel


def _make_agg(nch):
    ch_per_sc = nch // 2
    mesh = plsc.VectorSubcoreMesh(core_axis_name="c", subcore_axis_name="s")

    @functools.partial(
        pl.kernel,
        out_type=jax.ShapeDtypeStruct((N_PAD, nch * 128), jnp.float32),
        mesh=mesh,
        scratch_types=[
            pltpu.VMEM((KB, 128), jnp.int32),
            pltpu.VMEM((KB, 128), jnp.int32),
            pltpu.VMEM((128, 128), jnp.float32),
            pltpu.VMEM_SHARED((N_PAD, 128), jnp.float32),
        ],
    )
    def agg_kernel(y_hbm, src_hbm, dst_hbm, out_hbm, src_v, dst_v, rows_v, acc):
        cid = lax.axis_index("c")
        sid = lax.axis_index("s")
        r0 = sid * ROWS_T
        e0 = sid * ER_T

        for cc in range(ch_per_sc):
            ch = cid * ch_per_sc + cc
            ytab = y_hbm.at[ch]

            def ibody(p, c):
                pltpu.sync_copy(ytab.at[pl.ds(r0 + p * 128, 128)], rows_v)
                pltpu.sync_copy(rows_v, acc.at[pl.ds(r0 + p * 128, 128)])
                return c

            lax.fori_loop(0, ROWS_T // 128, ibody, 0)
            plsc.subcore_barrier()

            def gbody(g, c):
                pltpu.sync_copy(src_hbm.at[pl.ds(e0 + g * KB, KB)], src_v)
                pltpu.sync_copy(dst_hbm.at[pl.ds(e0 + g * KB, KB)], dst_v)

                def bbody(j, c2):
                    pltpu.sync_copy(ytab.at[src_v.at[j]], rows_v)
                    pltpu.sync_copy(rows_v, acc.at[dst_v.at[j]], add=True)
                    return c2

                lax.fori_loop(0, KB, bbody, 0)
                return c

            lax.fori_loop(0, ER_T // KB, gbody, 0)
            plsc.subcore_barrier()

            def obody(p, c):
                pltpu.sync_copy(acc.at[pl.ds(r0 + p * 128, 128)], rows_v)
                pltpu.sync_copy(
                    rows_v,
                    out_hbm.at[pl.ds(r0 + p * 128, 128),
                               pl.ds(ch * 128, 128)],
                )
                return c

            lax.fori_loop(0, ROWS_T // 128, obody, 0)

    return agg_kernel


_deg_kernel = _make_deg()
_agg4 = _make_agg(4)
_agg2 = _make_agg(2)


def _dis_body(p_ref, o_ref):
    deg = p_ref[0:1, :] + p_ref[1:2, :] + 1.0
    o_ref[...] = jnp.where(
        deg > 0.0, lax.rsqrt(jnp.maximum(deg, 1e-12)), 0.0)


def _dis_from_parts(parts):
    return pl.pallas_call(
        _dis_body,
        grid=(8,),
        in_specs=[pl.BlockSpec((2, N_PAD // 8), lambda i: (0, i))],
        out_specs=pl.BlockSpec((1, N_PAD // 8), lambda i: (0, i)),
        out_shape=jax.ShapeDtypeStruct((1, N_PAD), jnp.float32),
    )(parts)


def _k1_body(x_ref, dis_ref, w_ref, y_ref):
    y_ref[0] = jnp.dot(x_ref[...], w_ref[...],
                       preferred_element_type=jnp.float32) * dis_ref[...]


def _first_layer(xp, dis_col, W1):
    nch = H // 128
    return pl.pallas_call(
        _k1_body,
        grid=(N_PAD // BN, nch),
        in_specs=[
            pl.BlockSpec((BN, D_IN), lambda i, j: (i, 0)),
            pl.BlockSpec((BN, 1), lambda i, j: (i, 0)),
            pl.BlockSpec((D_IN, 128), lambda i, j: (0, j)),
        ],
        out_specs=pl.BlockSpec((1, BN, 128), lambda i, j: (j, i, 0)),
        out_shape=jax.ShapeDtypeStruct((nch, N_PAD, 128), jnp.float32),
    )(xp, dis_col, W1)


def _mid_body(s_ref, dis_ref, b_ref, g_ref, be_ref, w_ref, y_ref):
    t = s_ref[...] * dis_ref[...] + b_ref[...]
    mu = jnp.mean(t, axis=1, keepdims=True)
    d = t - mu
    var = jnp.mean(d * d, axis=1, keepdims=True)
    h = d * lax.rsqrt(var + 1e-5) * g_ref[...] + be_ref[...]
    h = jnp.maximum(h, 0.0)
    y = jnp.dot(h, w_ref[...], preferred_element_type=jnp.float32)
    y_ref[0] = y * dis_ref[...]


def _mid_layer(s, dis_col, b, g, be, W):
    nch = W.shape[1] // 128
    return pl.pallas_call(
        _mid_body,
        grid=(N_PAD // BN, nch),
        in_specs=[
            pl.BlockSpec((BN, H), lambda i, j: (i, 0)),
            pl.BlockSpec((BN, 1), lambda i, j: (i, 0)),
            pl.BlockSpec((1, H), lambda i, j: (0, 0)),
            pl.BlockSpec((1, H), lambda i, j: (0, 0)),
            pl.BlockSpec((1, H), lambda i, j: (0, 0)),
            pl.BlockSpec((H, 128), lambda i, j: (0, j)),
        ],
        out_specs=pl.BlockSpec((1, BN, 128), lambda i, j: (j, i, 0)),
        out_shape=jax.ShapeDtypeStruct((nch, N_PAD, 128), jnp.float32),
    )(s, dis_col, b, g, be, W)


def _final_body(s_ref, dis_ref, b_ref, g_ref, be_ref, o_ref):
    t = s_ref[...] * dis_ref[...] + b_ref[...]
    mu = jnp.mean(t, axis=1, keepdims=True)
    d = t - mu
    var = jnp.mean(d * d, axis=1, keepdims=True)
    o_ref[...] = d * lax.rsqrt(var + 1e-5) * g_ref[...] + be_ref[...]


def _final_layer(s, dis_col, b, g, be):
    return pl.pallas_call(
        _final_body,
        grid=(N_PAD // BN,),
        in_specs=[
            pl.BlockSpec((BN, D_OUT), lambda i: (i, 0)),
            pl.BlockSpec((BN, 1), lambda i: (i, 0)),
            pl.BlockSpec((1, D_OUT), lambda i: (0, 0)),
            pl.BlockSpec((1, D_OUT), lambda i: (0, 0)),
            pl.BlockSpec((1, D_OUT), lambda i: (0, 0)),
        ],
        out_specs=pl.BlockSpec((BN, D_OUT), lambda i: (i, 0)),
        out_shape=jax.ShapeDtypeStruct((N_PAD, D_OUT), jnp.float32),
    )(s, dis_col, b, g, be)


def kernel(x, edge_index, W1, b1, g1, be1, W2, b2, g2, be2,
           W3, b3, g3, be3, W4, b4, g4, be4):
    xp = jnp.pad(x, ((0, N_PAD - N), (0, 0)))
    src = jnp.pad(edge_index[0], (0, E_PAD - E), constant_values=PAD_IDX)
    dst = jnp.pad(edge_index[1], (0, E_PAD - E), constant_values=PAD_IDX)
    src2 = src.reshape(EROWS, 128)
    dst2 = dst.reshape(EROWS, 128)

    parts = _deg_kernel(dst)
    dis_col = _dis_from_parts(parts).reshape(N_PAD, 1)

    y1 = _first_layer(xp, dis_col, W1)
    s1 = _agg4(y1, src2, dst2)
    y2 = _mid_layer(s1, dis_col, b1.reshape(1, H), g1.reshape(1, H),
                    be1.reshape(1, H), W2)
    s2 = _agg4(y2, src2, dst2)
    y3 = _mid_layer(s2, dis_col, b2.reshape(1, H), g2.reshape(1, H),
                    be2.reshape(1, H), W3)
    s3 = _agg4(y3, src2, dst2)
    y4 = _mid_layer(s3, dis_col, b3.reshape(1, H), g3.reshape(1, H),
                    be3.reshape(1, H), W4)
    s4 = _agg2(y4, src2, dst2)
    out = _final_layer(s4, dis_col, b4.reshape(1, D_OUT),
                       g4.reshape(1, D_OUT), be4.reshape(1, D_OUT))
    return out[:N]
